# baseline (device time: 173573 ns/iter reference)
import jax
import jax.numpy as jnp
from jax import lax
from jax.experimental import pallas as pl
from jax.experimental.pallas import tpu as pltpu

N_DEV = 4
SQ = 1024
SKV_LOCAL = 1024
HQ_LOCAL = 8
DH = 128
D_MODEL = 1024
QCHUNK = 256
QROWS = 256
SCALE = 0.08838834764831843
MESH = pl.DeviceIdType.MESH


def _barrier(my_pos):
    bsem = pltpu.get_barrier_semaphore()
    for d in range(1, N_DEV):
        pl.semaphore_signal(
            bsem, inc=1, device_id=((my_pos + d) % N_DEV,), device_id_type=MESH
        )
    pl.semaphore_wait(bsem, N_DEV - 1)


def _fused_body(
    kt_ref, vt_ref, q_ref, out_ref,
    kbuf, vbuf, bias_ref, acc_ref, l_ref,
    ksend, krecv, vsend, vrecv,
):
    my = lax.axis_index("i")
    _barrier(my)

    sends = []
    for h in range(HQ_LOCAL):
        for d in (1, 3, 2):
            dest = (my + d) % N_DEV
            slot = N_DEV - d - 1
            sem = (d - 1) * HQ_LOCAL + h
            rk = pltpu.make_async_remote_copy(
                src_ref=kt_ref.at[dest * HQ_LOCAL + h],
                dst_ref=kbuf.at[slot, h],
                send_sem=ksend.at[sem],
                recv_sem=krecv.at[slot * HQ_LOCAL + h],
                device_id=(dest,),
                device_id_type=MESH,
            )
            rv = pltpu.make_async_remote_copy(
                src_ref=vt_ref.at[dest * HQ_LOCAL + h],
                dst_ref=vbuf.at[slot, h],
                send_sem=vsend.at[sem],
                recv_sem=vrecv.at[slot * HQ_LOCAL + h],
                device_id=(dest,),
                device_id_type=MESH,
            )
            rk.start()
            rv.start()
            sends.append((rk, rv))

    for e in range(N_DEV):
        src = (my + e) % N_DEV
        for qc in range(SQ // QCHUNK):
            qb = (
                qc * QCHUNK
                + lax.broadcasted_iota(jnp.int32, (QCHUNK, SKV_LOCAL), 0)
            ) // 64
            kb = src * 16 + lax.broadcasted_iota(
                jnp.int32, (QCHUNK, SKV_LOCAL), 1
            ) // 64
            m = (qb == kb) | (kb == 0) | ((qb + kb) % 3 == 0)
            bias_ref[
                qc * QCHUNK : (qc + 1) * QCHUNK,
                e * SKV_LOCAL : (e + 1) * SKV_LOCAL,
            ] = jnp.where(m, 0.0, -1e9).astype(jnp.bfloat16)

    def process(e, h, kh, vh):
        def qbody(qi, _):
            rows = pl.ds(qi * QROWS, QROWS)
            qh = q_ref[rows, h * DH : (h + 1) * DH]
            s = lax.dot_general(
                qh, kh, (((1,), (1,)), ((), ())),
                preferred_element_type=jnp.float32,
            )
            s = s * SCALE + bias_ref[
                rows, e * SKV_LOCAL : (e + 1) * SKV_LOCAL
            ].astype(jnp.float32)
            p = jnp.exp(s)
            lsum = jnp.sum(p, axis=1)
            pv = lax.dot_general(
                p.astype(jnp.bfloat16), vh, (((1,), (0,)), ((), ())),
                preferred_element_type=jnp.float32,
            )
            if e == 0:
                acc_ref[h, rows, :] = pv
                l_ref[h, rows] = lsum
            else:
                acc_ref[h, rows, :] = acc_ref[h, rows, :] + pv
                l_ref[h, rows] = l_ref[h, rows] + lsum
            return 0

        lax.fori_loop(0, SQ // QROWS, qbody, 0)

    for h in range(HQ_LOCAL):
        process(
            0, h,
            kt_ref[my * HQ_LOCAL + h],
            vt_ref[my * HQ_LOCAL + h].astype(jnp.bfloat16),
        )

    for h in range(HQ_LOCAL):
        for e in (1, 3, 2):
            src = (my + e) % N_DEV
            slot = e - 1
            wk = pltpu.make_async_remote_copy(
                src_ref=kt_ref.at[0],
                dst_ref=kbuf.at[slot, h],
                send_sem=ksend.at[0],
                recv_sem=krecv.at[slot * HQ_LOCAL + h],
                device_id=(src,),
                device_id_type=MESH,
            )
            wv = pltpu.make_async_remote_copy(
                src_ref=vt_ref.at[0],
                dst_ref=vbuf.at[slot, h],
                send_sem=vsend.at[0],
                recv_sem=vrecv.at[slot * HQ_LOCAL + h],
                device_id=(src,),
                device_id_type=MESH,
            )
            wk.wait_recv()
            wv.wait_recv()
            process(
                e, h, kbuf[slot, h], vbuf[slot, h].astype(jnp.bfloat16)
            )

    for h in range(HQ_LOCAL):
        ctx = acc_ref[h] / l_ref[h][:, None]
        out_ref[:, h * DH : (h + 1) * DH] = ctx.astype(jnp.bfloat16)

    for rk, rv in sends:
        rk.wait_send()
        rv.wait_send()


QSEG = SQ // N_DEV


def _proj_ar_body(
    ctx_ref, wo_ref, out_ref, pbuf, rsbuf, agbuf,
    rs_send, rs_recv, ag_send, ag_recv,
):
    my = lax.axis_index("i")
    _barrier(my)

    pbuf[...] = jnp.dot(
        ctx_ref[...], wo_ref[...], preferred_element_type=jnp.float32
    ).astype(jnp.bfloat16)

    sends = []
    for d in range(1, N_DEV):
        dest = (my + d) % N_DEV
        slot = N_DEV - d - 1
        r = pltpu.make_async_remote_copy(
            src_ref=pbuf.at[pl.ds(dest * QSEG, QSEG), :],
            dst_ref=rsbuf.at[slot],
            send_sem=rs_send.at[d],
            recv_sem=rs_recv.at[slot],
            device_id=(dest,),
            device_id_type=MESH,
        )
        r.start()
        sends.append(r)

    seg = pbuf[pl.ds(my * QSEG, QSEG), :].astype(jnp.float32)
    for e in (1, 3, 2):
        slot = e - 1
        wr = pltpu.make_async_remote_copy(
            src_ref=pbuf.at[pl.ds(0, QSEG), :],
            dst_ref=rsbuf.at[slot],
            send_sem=rs_send.at[0],
            recv_sem=rs_recv.at[slot],
            device_id=((my + e) % N_DEV,),
            device_id_type=MESH,
        )
        wr.wait_recv()
        seg = seg + rsbuf[slot].astype(jnp.float32)

    agbuf[my] = seg.astype(jnp.bfloat16)
    out_ref[pl.ds(my * QSEG, QSEG), :] = seg
    for d in range(1, N_DEV):
        dest = (my + d) % N_DEV
        r = pltpu.make_async_remote_copy(
            src_ref=agbuf.at[my],
            dst_ref=agbuf.at[my],
            send_sem=ag_send.at[d],
            recv_sem=ag_recv.at[my],
            device_id=(dest,),
            device_id_type=MESH,
        )
        r.start()
        sends.append(r)

    for e in (1, 3, 2):
        src = (my + e) % N_DEV
        wr = pltpu.make_async_remote_copy(
            src_ref=agbuf.at[0],
            dst_ref=agbuf.at[src],
            send_sem=ag_send.at[0],
            recv_sem=ag_recv.at[src],
            device_id=(src,),
            device_id_type=MESH,
        )
        wr.wait_recv()
        out_ref[pl.ds(src * QSEG, QSEG), :] = agbuf[src].astype(jnp.float32)

    for r in sends:
        r.wait_send()


def kernel(x, Wq, K_ext, V_ext, Wo):
    bf16 = jnp.bfloat16
    xb = x[0].astype(bf16)
    wqb = Wq.astype(bf16)
    wob = Wo.astype(bf16)
    qb = jnp.dot(xb, wqb, preferred_element_type=jnp.float32).astype(bf16)
    f8 = jnp.float8_e4m3fn
    kt = jnp.transpose(K_ext[0], (1, 0, 2)).astype(bf16)
    vt = jnp.transpose(V_ext[0], (1, 0, 2)).astype(f8)

    ctx = pl.pallas_call(
        _fused_body,
        out_shape=jax.ShapeDtypeStruct((SQ, D_MODEL), bf16),
        in_specs=[pl.BlockSpec(memory_space=pltpu.VMEM)] * 3,
        out_specs=pl.BlockSpec(memory_space=pltpu.VMEM),
        scratch_shapes=[
            pltpu.VMEM((N_DEV - 1, HQ_LOCAL, SKV_LOCAL, DH), bf16),
            pltpu.VMEM((N_DEV - 1, HQ_LOCAL, SKV_LOCAL, DH), f8),
            pltpu.VMEM((SQ, N_DEV * SKV_LOCAL), bf16),
            pltpu.VMEM((HQ_LOCAL, SQ, DH), jnp.float32),
            pltpu.VMEM((HQ_LOCAL, SQ), jnp.float32),
            pltpu.SemaphoreType.DMA(((N_DEV - 1) * HQ_LOCAL,)),
            pltpu.SemaphoreType.DMA(((N_DEV - 1) * HQ_LOCAL,)),
            pltpu.SemaphoreType.DMA(((N_DEV - 1) * HQ_LOCAL,)),
            pltpu.SemaphoreType.DMA(((N_DEV - 1) * HQ_LOCAL,)),
        ],
        compiler_params=pltpu.CompilerParams(
            collective_id=0, vmem_limit_bytes=44 * 1024 * 1024
        ),
    )(kt, vt, qb)

    out2d = pl.pallas_call(
        _proj_ar_body,
        out_shape=jax.ShapeDtypeStruct((SQ, D_MODEL), jnp.float32),
        in_specs=[pl.BlockSpec(memory_space=pltpu.VMEM)] * 2,
        out_specs=pl.BlockSpec(memory_space=pltpu.VMEM),
        scratch_shapes=[
            pltpu.VMEM((SQ, D_MODEL), bf16),
            pltpu.VMEM((N_DEV - 1, QSEG, D_MODEL), bf16),
            pltpu.VMEM((N_DEV, QSEG, D_MODEL), bf16),
            pltpu.SemaphoreType.DMA((N_DEV,)),
            pltpu.SemaphoreType.DMA((N_DEV - 1,)),
            pltpu.SemaphoreType.DMA((N_DEV,)),
            pltpu.SemaphoreType.DMA((N_DEV,)),
        ],
        compiler_params=pltpu.CompilerParams(collective_id=1),
    )(ctx, wob)

    return out2d.reshape(1, SQ, D_MODEL)


# device time: 166686 ns/iter; 1.0413x vs baseline; 1.0413x over previous
import jax
import jax.numpy as jnp
from jax import lax
from jax.experimental import pallas as pl
from jax.experimental.pallas import tpu as pltpu

N_DEV = 4
SQ = 1024
SKV_LOCAL = 1024
HQ_LOCAL = 8
DH = 128
D_MODEL = 1024
QCHUNK = 256
QROWS = 256
SCALE = 0.08838834764831843
MESH = pl.DeviceIdType.MESH


def _barrier(my_pos):
    bsem = pltpu.get_barrier_semaphore()
    for d in range(1, N_DEV):
        pl.semaphore_signal(
            bsem, inc=1, device_id=((my_pos + d) % N_DEV,), device_id_type=MESH
        )
    pl.semaphore_wait(bsem, N_DEV - 1)


def _fused_body(
    kt_ref, vt_ref, q_ref, out_ref,
    kbuf, vbuf, bias_ref, acc_ref, l_ref,
    ksend, krecv, vsend, vrecv,
):
    my = lax.axis_index("i")
    _barrier(my)

    sends = []
    for h in range(HQ_LOCAL):
        for d in (1, 3, 2):
            dest = (my + d) % N_DEV
            slot = N_DEV - d - 1
            sem = (d - 1) * HQ_LOCAL + h
            rk = pltpu.make_async_remote_copy(
                src_ref=kt_ref.at[dest * HQ_LOCAL + h],
                dst_ref=kbuf.at[slot, h],
                send_sem=ksend.at[sem],
                recv_sem=krecv.at[slot * HQ_LOCAL + h],
                device_id=(dest,),
                device_id_type=MESH,
            )
            rv = pltpu.make_async_remote_copy(
                src_ref=vt_ref.at[dest * HQ_LOCAL + h],
                dst_ref=vbuf.at[slot, h],
                send_sem=vsend.at[sem],
                recv_sem=vrecv.at[slot * HQ_LOCAL + h],
                device_id=(dest,),
                device_id_type=MESH,
            )
            rk.start()
            rv.start()
            sends.append((rk, rv))

    for e in range(N_DEV):
        src = (my + e) % N_DEV
        for qc in range(SQ // QCHUNK):
            qb = (
                qc * QCHUNK
                + lax.broadcasted_iota(jnp.int32, (QCHUNK, SKV_LOCAL), 0)
            ) // 64
            kb = src * 16 + lax.broadcasted_iota(
                jnp.int32, (QCHUNK, SKV_LOCAL), 1
            ) // 64
            m = (qb == kb) | (kb == 0) | ((qb + kb) % 3 == 0)
            bias_ref[
                qc * QCHUNK : (qc + 1) * QCHUNK,
                e * SKV_LOCAL : (e + 1) * SKV_LOCAL,
            ] = jnp.where(m, 0.0, -1e9).astype(jnp.bfloat16)

    def process(e, h, kh, vh):
        def qbody(qi, _):
            rows = pl.ds(qi * QROWS, QROWS)
            qh = q_ref[rows, h * DH : (h + 1) * DH]
            s = lax.dot_general(
                qh, kh, (((1,), (1,)), ((), ())),
                preferred_element_type=jnp.float32,
            )
            s = s * SCALE + bias_ref[
                rows, e * SKV_LOCAL : (e + 1) * SKV_LOCAL
            ].astype(jnp.float32)
            p = jnp.exp(s)
            lsum = jnp.sum(p, axis=1)
            pv = lax.dot_general(
                p.astype(jnp.bfloat16), vh, (((1,), (0,)), ((), ())),
                preferred_element_type=jnp.float32,
            )
            if e == 0:
                acc_ref[h, rows, :] = pv
                l_ref[h, rows] = lsum
            else:
                acc_ref[h, rows, :] = acc_ref[h, rows, :] + pv
                l_ref[h, rows] = l_ref[h, rows] + lsum
            return 0

        lax.fori_loop(0, SQ // QROWS, qbody, 0)

    for h in range(HQ_LOCAL):
        process(0, h, kt_ref[my * HQ_LOCAL + h], vt_ref[my * HQ_LOCAL + h])

    for h in range(HQ_LOCAL):
        for e in (1, 3, 2):
            src = (my + e) % N_DEV
            slot = e - 1
            wk = pltpu.make_async_remote_copy(
                src_ref=kt_ref.at[0],
                dst_ref=kbuf.at[slot, h],
                send_sem=ksend.at[0],
                recv_sem=krecv.at[slot * HQ_LOCAL + h],
                device_id=(src,),
                device_id_type=MESH,
            )
            wv = pltpu.make_async_remote_copy(
                src_ref=vt_ref.at[0],
                dst_ref=vbuf.at[slot, h],
                send_sem=vsend.at[0],
                recv_sem=vrecv.at[slot * HQ_LOCAL + h],
                device_id=(src,),
                device_id_type=MESH,
            )
            wk.wait_recv()
            wv.wait_recv()
            process(e, h, kbuf[slot, h], vbuf[slot, h])

    for h in range(HQ_LOCAL):
        ctx = acc_ref[h] / l_ref[h][:, None]
        out_ref[:, h * DH : (h + 1) * DH] = ctx.astype(jnp.bfloat16)

    for rk, rv in sends:
        rk.wait_send()
        rv.wait_send()


QSEG = SQ // N_DEV


def _proj_ar_body(
    ctx_ref, wo_ref, out_ref, pbuf, rsbuf, agbuf,
    rs_send, rs_recv, ag_send, ag_recv,
):
    my = lax.axis_index("i")
    _barrier(my)

    sends = []
    for q in range(N_DEV):
        rows = slice(q * QSEG, (q + 1) * QSEG)
        pbuf[rows, :] = jnp.dot(
            ctx_ref[rows, :], wo_ref[...], preferred_element_type=jnp.float32
        ).astype(jnp.bfloat16)

        @pl.when(q != my)
        def _(q=q):
            slot = (my - q) % N_DEV - 1
            r = pltpu.make_async_remote_copy(
                src_ref=pbuf.at[pl.ds(q * QSEG, QSEG), :],
                dst_ref=rsbuf.at[slot],
                send_sem=rs_send.at[q],
                recv_sem=rs_recv.at[slot],
                device_id=(q,),
                device_id_type=MESH,
            )
            r.start()

    seg = pbuf[pl.ds(my * QSEG, QSEG), :].astype(jnp.float32)
    for e in (1, 3, 2):
        slot = e - 1
        wr = pltpu.make_async_remote_copy(
            src_ref=pbuf.at[pl.ds(0, QSEG), :],
            dst_ref=rsbuf.at[slot],
            send_sem=rs_send.at[0],
            recv_sem=rs_recv.at[slot],
            device_id=((my + e) % N_DEV,),
            device_id_type=MESH,
        )
        wr.wait_recv()
        seg = seg + rsbuf[slot].astype(jnp.float32)

    agbuf[my] = seg.astype(jnp.bfloat16)
    out_ref[pl.ds(my * QSEG, QSEG), :] = seg
    for d in range(1, N_DEV):
        dest = (my + d) % N_DEV
        r = pltpu.make_async_remote_copy(
            src_ref=agbuf.at[my],
            dst_ref=agbuf.at[my],
            send_sem=ag_send.at[d],
            recv_sem=ag_recv.at[my],
            device_id=(dest,),
            device_id_type=MESH,
        )
        r.start()
        sends.append(r)

    for e in (1, 3, 2):
        src = (my + e) % N_DEV
        wr = pltpu.make_async_remote_copy(
            src_ref=agbuf.at[0],
            dst_ref=agbuf.at[src],
            send_sem=ag_send.at[0],
            recv_sem=ag_recv.at[src],
            device_id=(src,),
            device_id_type=MESH,
        )
        wr.wait_recv()
        out_ref[pl.ds(src * QSEG, QSEG), :] = agbuf[src].astype(jnp.float32)

    for q in range(N_DEV):
        @pl.when(q != my)
        def _(q=q):
            slot = (my - q) % N_DEV - 1
            r = pltpu.make_async_remote_copy(
                src_ref=pbuf.at[pl.ds(q * QSEG, QSEG), :],
                dst_ref=rsbuf.at[slot],
                send_sem=rs_send.at[q],
                recv_sem=rs_recv.at[slot],
                device_id=(q,),
                device_id_type=MESH,
            )
            r.wait_send()

    for r in sends:
        r.wait_send()


def kernel(x, Wq, K_ext, V_ext, Wo):
    bf16 = jnp.bfloat16
    xb = x[0].astype(bf16)
    wqb = Wq.astype(bf16)
    wob = Wo.astype(bf16)
    qb = jnp.dot(xb, wqb, preferred_element_type=jnp.float32).astype(bf16)
    kt = jnp.transpose(K_ext[0], (1, 0, 2)).astype(bf16)
    vt = jnp.transpose(V_ext[0], (1, 0, 2)).astype(bf16)

    ctx = pl.pallas_call(
        _fused_body,
        out_shape=jax.ShapeDtypeStruct((SQ, D_MODEL), bf16),
        in_specs=[pl.BlockSpec(memory_space=pltpu.VMEM)] * 3,
        out_specs=pl.BlockSpec(memory_space=pltpu.VMEM),
        scratch_shapes=[
            pltpu.VMEM((N_DEV - 1, HQ_LOCAL, SKV_LOCAL, DH), bf16),
            pltpu.VMEM((N_DEV - 1, HQ_LOCAL, SKV_LOCAL, DH), bf16),
            pltpu.VMEM((SQ, N_DEV * SKV_LOCAL), bf16),
            pltpu.VMEM((HQ_LOCAL, SQ, DH), jnp.float32),
            pltpu.VMEM((HQ_LOCAL, SQ), jnp.float32),
            pltpu.SemaphoreType.DMA(((N_DEV - 1) * HQ_LOCAL,)),
            pltpu.SemaphoreType.DMA(((N_DEV - 1) * HQ_LOCAL,)),
            pltpu.SemaphoreType.DMA(((N_DEV - 1) * HQ_LOCAL,)),
            pltpu.SemaphoreType.DMA(((N_DEV - 1) * HQ_LOCAL,)),
        ],
        compiler_params=pltpu.CompilerParams(
            collective_id=0, vmem_limit_bytes=44 * 1024 * 1024
        ),
    )(kt, vt, qb)

    out2d = pl.pallas_call(
        _proj_ar_body,
        out_shape=jax.ShapeDtypeStruct((SQ, D_MODEL), jnp.float32),
        in_specs=[pl.BlockSpec(memory_space=pltpu.VMEM)] * 2,
        out_specs=pl.BlockSpec(memory_space=pltpu.VMEM),
        scratch_shapes=[
            pltpu.VMEM((SQ, D_MODEL), bf16),
            pltpu.VMEM((N_DEV - 1, QSEG, D_MODEL), bf16),
            pltpu.VMEM((N_DEV, QSEG, D_MODEL), bf16),
            pltpu.SemaphoreType.DMA((N_DEV,)),
            pltpu.SemaphoreType.DMA((N_DEV - 1,)),
            pltpu.SemaphoreType.DMA((N_DEV,)),
            pltpu.SemaphoreType.DMA((N_DEV,)),
        ],
        compiler_params=pltpu.CompilerParams(collective_id=1),
    )(ctx, wob)

    return out2d.reshape(1, SQ, D_MODEL)


# device time: 165700 ns/iter; 1.0475x vs baseline; 1.0060x over previous
import jax
import jax.numpy as jnp
from jax import lax
from jax.experimental import pallas as pl
from jax.experimental.pallas import tpu as pltpu

N_DEV = 4
SQ = 1024
SKV_LOCAL = 1024
HQ_LOCAL = 8
DH = 128
D_MODEL = 1024
QCHUNK = 256
QROWS = 256
SCALE = 0.08838834764831843
MESH = pl.DeviceIdType.MESH


def _barrier(my_pos):
    bsem = pltpu.get_barrier_semaphore()
    for d in range(1, N_DEV):
        pl.semaphore_signal(
            bsem, inc=1, device_id=((my_pos + d) % N_DEV,), device_id_type=MESH
        )
    pl.semaphore_wait(bsem, N_DEV - 1)


def _fused_body(
    kt_ref, vt_ref, q_ref, out_ref,
    kbuf, vbuf, bias_ref, acc_ref, l_ref,
    ksend, krecv, vsend, vrecv,
):
    my = lax.axis_index("i")
    _barrier(my)

    sends = []
    for h in range(HQ_LOCAL):
        for d in (1, 3, 2):
            dest = (my + d) % N_DEV
            slot = N_DEV - d - 1
            sem = (d - 1) * HQ_LOCAL + h
            rk = pltpu.make_async_remote_copy(
                src_ref=kt_ref.at[dest * HQ_LOCAL + h],
                dst_ref=kbuf.at[slot, h],
                send_sem=ksend.at[sem],
                recv_sem=krecv.at[slot * HQ_LOCAL + h],
                device_id=(dest,),
                device_id_type=MESH,
            )
            rv = pltpu.make_async_remote_copy(
                src_ref=vt_ref.at[dest * HQ_LOCAL + h],
                dst_ref=vbuf.at[slot, h],
                send_sem=vsend.at[sem],
                recv_sem=vrecv.at[slot * HQ_LOCAL + h],
                device_id=(dest,),
                device_id_type=MESH,
            )
            rk.start()
            rv.start()
            sends.append((rk, rv))

    for e in range(N_DEV):
        src = (my + e) % N_DEV
        for qc in range(SQ // QCHUNK):
            qb = (
                qc * QCHUNK
                + lax.broadcasted_iota(jnp.int32, (QCHUNK, SKV_LOCAL), 0)
            ) // 64
            kb = src * 16 + lax.broadcasted_iota(
                jnp.int32, (QCHUNK, SKV_LOCAL), 1
            ) // 64
            m = (qb == kb) | (kb == 0) | ((qb + kb) % 3 == 0)
            bias_ref[
                qc * QCHUNK : (qc + 1) * QCHUNK,
                e * SKV_LOCAL : (e + 1) * SKV_LOCAL,
            ] = jnp.where(m, 0.0, -1e9).astype(jnp.bfloat16)

    def process(e, h, kh, vh):
        def qbody(qi, _):
            rows = pl.ds(qi * QROWS, QROWS)
            qh = q_ref[rows, h * DH : (h + 1) * DH]
            s = lax.dot_general(
                qh, kh, (((1,), (1,)), ((), ())),
                preferred_element_type=jnp.float32,
            )
            s = s * SCALE + bias_ref[
                rows, e * SKV_LOCAL : (e + 1) * SKV_LOCAL
            ].astype(jnp.float32)
            p = jnp.exp(s)
            lsum = jnp.sum(p, axis=1)
            pv = lax.dot_general(
                p.astype(jnp.bfloat16), vh, (((1,), (0,)), ((), ())),
                preferred_element_type=jnp.float32,
            )
            if e == 0:
                acc_ref[h, rows, :] = pv
                l_ref[h, rows] = lsum
            else:
                acc_ref[h, rows, :] = acc_ref[h, rows, :] + pv
                l_ref[h, rows] = l_ref[h, rows] + lsum
            return 0

        lax.fori_loop(0, SQ // QROWS, qbody, 0)

    for h in range(HQ_LOCAL):
        process(0, h, kt_ref[my * HQ_LOCAL + h], vt_ref[my * HQ_LOCAL + h])

    for h in range(HQ_LOCAL):
        for e in (1, 3, 2):
            src = (my + e) % N_DEV
            slot = e - 1
            wk = pltpu.make_async_remote_copy(
                src_ref=kt_ref.at[0],
                dst_ref=kbuf.at[slot, h],
                send_sem=ksend.at[0],
                recv_sem=krecv.at[slot * HQ_LOCAL + h],
                device_id=(src,),
                device_id_type=MESH,
            )
            wv = pltpu.make_async_remote_copy(
                src_ref=vt_ref.at[0],
                dst_ref=vbuf.at[slot, h],
                send_sem=vsend.at[0],
                recv_sem=vrecv.at[slot * HQ_LOCAL + h],
                device_id=(src,),
                device_id_type=MESH,
            )
            wk.wait_recv()
            wv.wait_recv()
            process(e, h, kbuf[slot, h], vbuf[slot, h])

    for h in range(HQ_LOCAL):
        ctx = acc_ref[h] / l_ref[h][:, None]
        out_ref[:, h * DH : (h + 1) * DH] = ctx.astype(jnp.bfloat16)

    for rk, rv in sends:
        rk.wait_send()
        rv.wait_send()


QSEG = SQ // N_DEV


def _proj_ar_body(
    ctx_ref, wo_ref, out_ref, pbuf, rsbuf, agbuf,
    rs_send, rs_recv, ag_send, ag_recv,
):
    my = lax.axis_index("i")
    _barrier(my)

    sends = []
    for q in range(N_DEV):
        rows = slice(q * QSEG, (q + 1) * QSEG)
        pbuf[rows, :] = jnp.dot(
            ctx_ref[rows, :], wo_ref[...], preferred_element_type=jnp.float32
        ).astype(jnp.bfloat16)

        @pl.when(q != my)
        def _(q=q):
            slot = (my - q) % N_DEV - 1
            r = pltpu.make_async_remote_copy(
                src_ref=pbuf.at[pl.ds(q * QSEG, QSEG), :],
                dst_ref=rsbuf.at[slot],
                send_sem=rs_send.at[q],
                recv_sem=rs_recv.at[slot],
                device_id=(q,),
                device_id_type=MESH,
            )
            r.start()

    seg = pbuf[pl.ds(my * QSEG, QSEG), :].astype(jnp.float32)
    for e in (1, 3, 2):
        slot = e - 1
        wr = pltpu.make_async_remote_copy(
            src_ref=pbuf.at[pl.ds(0, QSEG), :],
            dst_ref=rsbuf.at[slot],
            send_sem=rs_send.at[0],
            recv_sem=rs_recv.at[slot],
            device_id=((my + e) % N_DEV,),
            device_id_type=MESH,
        )
        wr.wait_recv()
        seg = seg + rsbuf[slot].astype(jnp.float32)

    agbuf[my] = seg.astype(jnp.bfloat16)
    out_ref[pl.ds(my * QSEG, QSEG), :] = seg.astype(jnp.bfloat16)
    for d in range(1, N_DEV):
        dest = (my + d) % N_DEV
        r = pltpu.make_async_remote_copy(
            src_ref=agbuf.at[my],
            dst_ref=agbuf.at[my],
            send_sem=ag_send.at[d],
            recv_sem=ag_recv.at[my],
            device_id=(dest,),
            device_id_type=MESH,
        )
        r.start()
        sends.append(r)

    for e in (1, 3, 2):
        src = (my + e) % N_DEV
        wr = pltpu.make_async_remote_copy(
            src_ref=agbuf.at[0],
            dst_ref=agbuf.at[src],
            send_sem=ag_send.at[0],
            recv_sem=ag_recv.at[src],
            device_id=(src,),
            device_id_type=MESH,
        )
        wr.wait_recv()
        out_ref[pl.ds(src * QSEG, QSEG), :] = agbuf[src]

    for q in range(N_DEV):
        @pl.when(q != my)
        def _(q=q):
            slot = (my - q) % N_DEV - 1
            r = pltpu.make_async_remote_copy(
                src_ref=pbuf.at[pl.ds(q * QSEG, QSEG), :],
                dst_ref=rsbuf.at[slot],
                send_sem=rs_send.at[q],
                recv_sem=rs_recv.at[slot],
                device_id=(q,),
                device_id_type=MESH,
            )
            r.wait_send()

    for r in sends:
        r.wait_send()


def kernel(x, Wq, K_ext, V_ext, Wo):
    bf16 = jnp.bfloat16
    xb = x[0].astype(bf16)
    wqb = Wq.astype(bf16)
    wob = Wo.astype(bf16)
    qb = jnp.dot(xb, wqb, preferred_element_type=jnp.float32).astype(bf16)
    kt = jnp.transpose(K_ext[0], (1, 0, 2)).astype(bf16)
    vt = jnp.transpose(V_ext[0], (1, 0, 2)).astype(bf16)

    ctx = pl.pallas_call(
        _fused_body,
        out_shape=jax.ShapeDtypeStruct((SQ, D_MODEL), bf16),
        in_specs=[pl.BlockSpec(memory_space=pltpu.VMEM)] * 3,
        out_specs=pl.BlockSpec(memory_space=pltpu.VMEM),
        scratch_shapes=[
            pltpu.VMEM((N_DEV - 1, HQ_LOCAL, SKV_LOCAL, DH), bf16),
            pltpu.VMEM((N_DEV - 1, HQ_LOCAL, SKV_LOCAL, DH), bf16),
            pltpu.VMEM((SQ, N_DEV * SKV_LOCAL), bf16),
            pltpu.VMEM((HQ_LOCAL, SQ, DH), jnp.float32),
            pltpu.VMEM((HQ_LOCAL, SQ), jnp.float32),
            pltpu.SemaphoreType.DMA(((N_DEV - 1) * HQ_LOCAL,)),
            pltpu.SemaphoreType.DMA(((N_DEV - 1) * HQ_LOCAL,)),
            pltpu.SemaphoreType.DMA(((N_DEV - 1) * HQ_LOCAL,)),
            pltpu.SemaphoreType.DMA(((N_DEV - 1) * HQ_LOCAL,)),
        ],
        compiler_params=pltpu.CompilerParams(
            collective_id=0, vmem_limit_bytes=44 * 1024 * 1024
        ),
    )(kt, vt, qb)

    out2d = pl.pallas_call(
        _proj_ar_body,
        out_shape=jax.ShapeDtypeStruct((SQ, D_MODEL), bf16),
        in_specs=[pl.BlockSpec(memory_space=pltpu.VMEM)] * 2,
        out_specs=pl.BlockSpec(memory_space=pltpu.VMEM),
        scratch_shapes=[
            pltpu.VMEM((SQ, D_MODEL), bf16),
            pltpu.VMEM((N_DEV - 1, QSEG, D_MODEL), bf16),
            pltpu.VMEM((N_DEV, QSEG, D_MODEL), bf16),
            pltpu.SemaphoreType.DMA((N_DEV,)),
            pltpu.SemaphoreType.DMA((N_DEV - 1,)),
            pltpu.SemaphoreType.DMA((N_DEV,)),
            pltpu.SemaphoreType.DMA((N_DEV,)),
        ],
        compiler_params=pltpu.CompilerParams(collective_id=1),
    )(ctx, wob)

    return out2d.reshape(1, SQ, D_MODEL)
